# baseline (speedup 1.0000x reference)
import jax
import jax.numpy as jnp
from jax import lax
from jax.experimental import pallas as pl
from jax.experimental.pallas import tpu as pltpu
from jax.experimental.pallas import tpu_sc as plsc

N = 10000
E = 160000
F = 256
FH = 128
NCLS = 64
NC, NS = 2, 16
NW = NC * NS
CH = 128
IBLK = 8
NPAD = 10240
DUMMY = N
EP = 163840
NCHUNK = EP // CH
SCHUNK = NCHUNK // NS
SCC = 64
SNCHUNK = EP // SCC
STILE = SNCHUNK // NS
SIB = 16
SNBUF = 4
ZROWS = NPAD // NS
HR = NPAD // 128
EBLK = 1024
EDT = EP // NS
ERT = EP // NW
BK = 2048

_mesh = plsc.VectorSubcoreMesh(core_axis_name="c", subcore_axis_name="s")
_no_layout = pltpu.CompilerParams(needs_layout_passes=False)


def _deg_body(src_hbm, dst_hbm, ident_hbm, zeros80_hbm, deg_hbm,
              deg_sh, hist_v, idx_v, ident_v):
    c = lax.axis_index("c")
    s = lax.axis_index("s")

    pltpu.sync_copy(zeros80_hbm, hist_v)

    rb = s * 8

    @pl.when(s < HR // 8)
    def _():
        pltpu.sync_copy(hist_v.at[pl.ds(0, 8)], deg_sh.at[pl.ds(rb, 8)])

    pltpu.sync_copy(ident_hbm, ident_v)
    plsc.subcore_barrier()

    ebase = s * EDT
    ones16 = jnp.full((16,), 1.0, jnp.float32)

    def scat(idx_hbm):
        def blk(b, _):
            pltpu.sync_copy(idx_hbm.at[pl.ds(ebase + b * EBLK, EBLK)], idx_v)
            def grp(g, _):
                idx16 = idx_v[pl.ds(g * 16, 16)]
                hi = jnp.right_shift(idx16, 7)
                lo = jnp.bitwise_and(idx16, 127)
                plsc.addupdate_scatter(hist_v, [hi, lo], ones16)
                return 0
            lax.fori_loop(0, EBLK // 16, grp, 0)
            return 0
        lax.fori_loop(0, EDT // EBLK, blk, 0)

    @pl.when(c == 0)
    def _():
        scat(src_hbm)

    @pl.when(c == 1)
    def _():
        scat(dst_hbm)

    pltpu.sync_copy(hist_v, deg_sh.at[ident_v], add=True)
    plsc.subcore_barrier()

    @pl.when(s < HR // 8)
    def _():
        pltpu.sync_copy(deg_sh.at[pl.ds(rb, 8)], hist_v.at[pl.ds(0, 8)])
        pltpu.sync_copy(hist_v.at[pl.ds(0, 8)], deg_hbm.at[c, pl.ds(rb, 8)])


@jax.jit
def _deg_call(src_flat, dst_flat, ident, zeros80):
    f = pl.kernel(
        _deg_body,
        out_type=jax.ShapeDtypeStruct((NC, HR, 128), jnp.float32),
        mesh=_mesh,
        compiler_params=_no_layout,
        scratch_types=[
            pltpu.VMEM_SHARED((HR, 128), jnp.float32),
            pltpu.VMEM((HR, 128), jnp.float32),
            pltpu.VMEM((EBLK,), jnp.int32),
            pltpu.VMEM((HR,), jnp.int32),
        ],
    )
    return f(src_flat, dst_flat, ident, zeros80)


def _norm_body(x_ref, dego_ref, degi_ref, xs0_ref, xs1_ref, ns_ref, nd_ref):
    ns = lax.rsqrt(jnp.clip(dego_ref[...], 1.0, None))
    nd = lax.rsqrt(jnp.clip(degi_ref[...], 1.0, None))
    ns_ref[...] = ns
    nd_ref[...] = nd
    xs = x_ref[...] * ns
    xs0_ref[...] = xs[:, :FH]
    xs1_ref[...] = xs[:, FH:]


_norm_call = pl.pallas_call(
    _norm_body,
    grid=(NPAD // BK,),
    in_specs=[
        pl.BlockSpec((BK, F), lambda i: (i, 0)),
        pl.BlockSpec((BK, 1), lambda i: (i, 0)),
        pl.BlockSpec((BK, 1), lambda i: (i, 0)),
    ],
    out_specs=[
        pl.BlockSpec((BK, FH), lambda i: (i, 0)),
        pl.BlockSpec((BK, FH), lambda i: (i, 0)),
        pl.BlockSpec((BK, 1), lambda i: (i, 0)),
        pl.BlockSpec((BK, 1), lambda i: (i, 0)),
    ],
    out_shape=[
        jax.ShapeDtypeStruct((NPAD, FH), jnp.float32),
        jax.ShapeDtypeStruct((NPAD, FH), jnp.float32),
        jax.ShapeDtypeStruct((NPAD, 1), jnp.float32),
        jax.ShapeDtypeStruct((NPAD, 1), jnp.float32),
    ],
)


def _agg_body(xs0_hbm, xs1_hbm, src_hbm, dst_hbm, zeros_hbm,
              s0_hbm, s1_hbm, S_sh, sidx_v, didx_v, sidx2_v, didx2_v,
              isem_a, isem_b, *ring):
    c = lax.axis_index("c")
    s = lax.axis_index("s")
    zb = s * ZROWS
    bufs = list(ring[:SNBUF])
    gsem = list(ring[SNBUF:2 * SNBUF])
    ssem = list(ring[2 * SNBUF:3 * SNBUF])
    g0 = bufs[0]

    pltpu.sync_copy(zeros_hbm, g0)

    def zero_step(k, _):
        pltpu.sync_copy(g0, S_sh.at[pl.ds(zb + k * SCC, SCC)])
        return 0
    lax.fori_loop(0, ZROWS // SCC, zero_step, 0)
    plsc.subcore_barrier()

    rowb = s * STILE
    NBLK = STILE // SIB

    def s_phase(xs_hbm):
        def drain_scat(b, rowref):
            pltpu.make_async_copy(bufs[b], S_sh.at[rowref], ssem[b]).wait()

        def drain_idx(isem, rows, sref, dref):
            pltpu.make_async_copy(src_hbm.at[rows], sref, isem).wait()
            pltpu.make_async_copy(dst_hbm.at[rows], dref, isem).wait()

        def blk_body(bid, scur, dcur, snext, dnext, isem_cur, isem_next,
                     drain_cur_pred, prefetch_pred):
            @pl.when(drain_cur_pred)
            def _():
                drain_idx(isem_cur, pl.ds(rowb + bid * SIB, SIB), scur, dcur)

            gd = [None] * SNBUF
            for k in range(SIB):
                b = k % SNBUF
                if k >= SNBUF:
                    drain_scat(b, dcur.at[0])
                else:
                    @pl.when(bid > 0)
                    def _(b=b):
                        drain_scat(b, dcur.at[0])
                if k == SNBUF:
                    @pl.when(prefetch_pred)
                    def _():
                        rows = pl.ds(rowb + (bid + 1) * SIB, SIB)
                        pltpu.async_copy(src_hbm.at[rows], snext, isem_next)
                        pltpu.async_copy(dst_hbm.at[rows], dnext, isem_next)
                gd[b] = pltpu.async_copy(xs_hbm.at[scur.at[k]], bufs[b], gsem[b])
                if k >= 1:
                    pb = (k - 1) % SNBUF
                    gd[pb].wait()
                    pltpu.async_copy(bufs[pb], S_sh.at[dcur.at[k - 1]],
                                     ssem[pb], add=True)
            lb = (SIB - 1) % SNBUF
            gd[lb].wait()
            pltpu.async_copy(bufs[lb], S_sh.at[dcur.at[SIB - 1]], ssem[lb],
                             add=True)

        pltpu.sync_copy(src_hbm.at[pl.ds(rowb, SIB)], sidx_v)
        pltpu.sync_copy(dst_hbm.at[pl.ds(rowb, SIB)], didx_v)

        def blkpair(bp, _):
            b0 = 2 * bp
            blk_body(b0, sidx_v, didx_v, sidx2_v, didx2_v, isem_a, isem_b,
                     bp > 0, b0 + 1 < NBLK)
            blk_body(b0 + 1, sidx2_v, didx2_v, sidx_v, didx_v, isem_b, isem_a,
                     b0 + 1 < NBLK, b0 + 2 < NBLK)
            return 0
        lax.fori_loop(0, NBLK // 2, blkpair, 0)
        for b in range(SNBUF):
            drain_scat(b, didx_v.at[0])

    @pl.when(c == 0)
    def _():
        s_phase(xs0_hbm)

    @pl.when(c == 1)
    def _():
        s_phase(xs1_hbm)

    plsc.subcore_barrier()

    def wout_step(k, _):
        pltpu.sync_copy(S_sh.at[pl.ds(zb + k * SCC, SCC)], g0)

        @pl.when(c == 0)
        def _():
            pltpu.sync_copy(g0, s0_hbm.at[pl.ds(zb + k * SCC, SCC)])

        @pl.when(c == 1)
        def _():
            pltpu.sync_copy(g0, s1_hbm.at[pl.ds(zb + k * SCC, SCC)])

        return 0
    lax.fori_loop(0, ZROWS // SCC, wout_step, 0)


@jax.jit
def _agg_call(xs0, xs1, src2d, dst2d, zeros64):
    f = pl.kernel(
        _agg_body,
        out_type=[
            jax.ShapeDtypeStruct((NPAD, FH), jnp.float32),
            jax.ShapeDtypeStruct((NPAD, FH), jnp.float32),
        ],
        mesh=_mesh,
        scratch_types=(
            [
                pltpu.VMEM_SHARED((NPAD, FH), jnp.float32),
                pltpu.VMEM((SIB, SCC), jnp.int32),
                pltpu.VMEM((SIB, SCC), jnp.int32),
                pltpu.VMEM((SIB, SCC), jnp.int32),
                pltpu.VMEM((SIB, SCC), jnp.int32),
                pltpu.SemaphoreType.DMA,
                pltpu.SemaphoreType.DMA,
            ]
            + [pltpu.VMEM((SCC, FH), jnp.float32)] * SNBUF
            + [pltpu.SemaphoreType.DMA] * (2 * SNBUF)
        ),
    )
    return f(xs0, xs1, src2d, dst2d, zeros64)


def _r_body(nd_hbm, src_hbm, dst_hbm, ident_hbm, zeros80_hbm, rp_hbm,
            r_sh, hist_v, nd_v, sidx_v, didx_v, ident_v):
    c = lax.axis_index("c")
    s = lax.axis_index("s")
    w = s * NC + c

    pltpu.sync_copy(zeros80_hbm, hist_v)

    rb = s * 8

    @pl.when(s < HR // 8)
    def _():
        pltpu.sync_copy(hist_v.at[pl.ds(0, 8)], r_sh.at[pl.ds(rb, 8)])

    pltpu.sync_copy(nd_hbm, nd_v)
    pltpu.sync_copy(ident_hbm, ident_v)
    plsc.subcore_barrier()

    ebase = w * ERT

    def blk(b, _):
        pltpu.sync_copy(src_hbm.at[pl.ds(ebase + b * EBLK, EBLK)], sidx_v)
        pltpu.sync_copy(dst_hbm.at[pl.ds(ebase + b * EBLK, EBLK)], didx_v)
        def grp(g, _):
            s16 = sidx_v[pl.ds(g * 16, 16)]
            d16 = didx_v[pl.ds(g * 16, 16)]
            vals = plsc.load_gather(nd_v, [d16])
            hi = jnp.right_shift(s16, 7)
            lo = jnp.bitwise_and(s16, 127)
            plsc.addupdate_scatter(hist_v, [hi, lo], vals)
            return 0
        lax.fori_loop(0, EBLK // 16, grp, 0)
        return 0
    lax.fori_loop(0, ERT // EBLK, blk, 0)

    pltpu.sync_copy(hist_v, r_sh.at[ident_v], add=True)
    plsc.subcore_barrier()

    @pl.when(s < HR // 8)
    def _():
        pltpu.sync_copy(r_sh.at[pl.ds(rb, 8)], hist_v.at[pl.ds(0, 8)])
        pltpu.sync_copy(hist_v.at[pl.ds(0, 8)], rp_hbm.at[c, pl.ds(rb, 8)])


@jax.jit
def _r_call(nd_flat, src_flat, dst_flat, ident, zeros80):
    f = pl.kernel(
        _r_body,
        out_type=jax.ShapeDtypeStruct((NC, HR, 128), jnp.float32),
        mesh=_mesh,
        compiler_params=_no_layout,
        scratch_types=[
            pltpu.VMEM_SHARED((HR, 128), jnp.float32),
            pltpu.VMEM((HR, 128), jnp.float32),
            pltpu.VMEM((NPAD,), jnp.float32),
            pltpu.VMEM((EBLK,), jnp.int32),
            pltpu.VMEM((EBLK,), jnp.int32),
            pltpu.VMEM((HR,), jnp.int32),
        ],
    )
    return f(nd_flat, src_flat, dst_flat, ident, zeros80)


def _final_body(s0_ref, s1_ref, nd_ref, ns_ref, rp0_ref, rp1_ref,
                w1_ref, b1_ref, w2_ref, b2_ref, out_ref, acc):
    i = pl.program_id(0)

    @pl.when(i == 0)
    def _():
        acc[...] = jnp.zeros_like(acc)

    sblk = jnp.concatenate([s0_ref[...], s1_ref[...]], axis=1)
    o = jnp.dot(sblk, w1_ref[...], preferred_element_type=jnp.float32)
    h = jnp.maximum(o * nd_ref[...] + b1_ref[...], 0.0)
    r = rp0_ref[...] + rp1_ref[...]
    cvec = ns_ref[...] * r
    rowid = i * BK + lax.broadcasted_iota(jnp.int32, (BK, 1), 0)
    cvec = jnp.where(rowid < N, cvec, 0.0)
    acc[...] += lax.dot_general(cvec, h, (((0,), (0,)), ((), ())),
                                preferred_element_type=jnp.float32)

    @pl.when(i == NPAD // BK - 1)
    def _():
        out_ref[...] = jnp.dot(acc[...] * (1.0 / N), w2_ref[...],
                               preferred_element_type=jnp.float32) + b2_ref[...]


_final_call = pl.pallas_call(
    _final_body,
    grid=(NPAD // BK,),
    in_specs=[
        pl.BlockSpec((BK, FH), lambda i: (i, 0)),
        pl.BlockSpec((BK, FH), lambda i: (i, 0)),
        pl.BlockSpec((BK, 1), lambda i: (i, 0)),
        pl.BlockSpec((BK, 1), lambda i: (i, 0)),
        pl.BlockSpec((BK, 1), lambda i: (i, 0)),
        pl.BlockSpec((BK, 1), lambda i: (i, 0)),
        pl.BlockSpec((F, F), lambda i: (0, 0)),
        pl.BlockSpec((1, F), lambda i: (0, 0)),
        pl.BlockSpec((F, NCLS), lambda i: (0, 0)),
        pl.BlockSpec((1, NCLS), lambda i: (0, 0)),
    ],
    out_specs=pl.BlockSpec((1, NCLS), lambda i: (0, 0)),
    out_shape=jax.ShapeDtypeStruct((1, NCLS), jnp.float32),
    scratch_shapes=[pltpu.VMEM((1, F), jnp.float32)],
)


def kernel(x, edge_index, W1, b1, W2, b2):
    src = edge_index[0].astype(jnp.int32)
    dst = edge_index[1].astype(jnp.int32)
    pad = jnp.full((EP - E,), DUMMY, jnp.int32)
    src_flat = jnp.concatenate([src, pad])
    dst_flat = jnp.concatenate([dst, pad])
    src2d = src_flat.reshape(SNCHUNK, SCC)
    dst2d = dst_flat.reshape(SNCHUNK, SCC)
    x_pad = jnp.pad(x, ((0, NPAD - N), (0, 0)))

    ident = jnp.arange(HR, dtype=jnp.int32)
    zeros64 = jnp.zeros((SCC, FH), jnp.float32)
    zeros80 = jnp.zeros((HR, 128), jnp.float32)

    degs = _deg_call(src_flat, dst_flat, ident, zeros80)
    dego_col = degs[0].reshape(NPAD, 1)
    degi_col = degs[1].reshape(NPAD, 1)
    xs0, xs1, ns_col, nd_col = _norm_call(x_pad, dego_col, degi_col)
    s0, s1 = _agg_call(xs0, xs1, src2d, dst2d, zeros64)
    rp = _r_call(nd_col.reshape(NPAD), src_flat, dst_flat, ident, zeros80)
    out = _final_call(s0, s1, nd_col, ns_col,
                      rp[0].reshape(NPAD, 1), rp[1].reshape(NPAD, 1),
                      W1, b1.reshape(1, F), W2, b2.reshape(1, NCLS))
    return out


# speedup vs baseline: 1.0518x; 1.0518x over previous
import jax
import jax.numpy as jnp
from jax import lax
from jax.experimental import pallas as pl
from jax.experimental.pallas import tpu as pltpu
from jax.experimental.pallas import tpu_sc as plsc

N = 10000
E = 160000
F = 256
FH = 128
NCLS = 64
NC, NS = 2, 16
NW = NC * NS
CH = 128
IBLK = 8
NPAD = 10240
DUMMY = N
EP = 163840
NCHUNK = EP // CH
SCHUNK = NCHUNK // NS
SCC = 64
SNCHUNK = EP // SCC
STILE = SNCHUNK // NS
SIB = 16
SNBUF = 4
ZROWS = NPAD // NS
HR = NPAD // 128
EBLK = 1024
EDT = EP // NS
ERT = EP // NW
BK = 2048

_mesh = plsc.VectorSubcoreMesh(core_axis_name="c", subcore_axis_name="s")
_no_layout = pltpu.CompilerParams(needs_layout_passes=False)


def _deg_body(src_hbm, dst_hbm, ident_hbm, deg_hbm,
              deg_sh, hist_v, idx_v, ident_v):
    c = lax.axis_index("c")
    s = lax.axis_index("s")

    def zrow(i, _):
        for g in range(8):
            hist_v[i, pl.ds(g * 16, 16)] = jnp.zeros((16,), jnp.float32)
        return 0
    lax.fori_loop(0, HR, zrow, 0)

    rb = s * 8

    @pl.when(s < HR // 8)
    def _():
        pltpu.sync_copy(hist_v.at[pl.ds(0, 8)], deg_sh.at[pl.ds(rb, 8)])

    pltpu.sync_copy(ident_hbm, ident_v)
    plsc.subcore_barrier()

    ebase = s * EDT
    ones16 = jnp.full((16,), 1.0, jnp.float32)

    def scat(idx_hbm):
        def blk(b, _):
            pltpu.sync_copy(idx_hbm.at[pl.ds(ebase + b * EBLK, EBLK)], idx_v)
            def grp(g, _):
                idx16 = idx_v[pl.ds(g * 16, 16)]
                hi = jnp.right_shift(idx16, 7)
                lo = jnp.bitwise_and(idx16, 127)
                plsc.addupdate_scatter(hist_v, [hi, lo], ones16)
                return 0
            lax.fori_loop(0, EBLK // 16, grp, 0)
            return 0
        lax.fori_loop(0, EDT // EBLK, blk, 0)

    @pl.when(c == 0)
    def _():
        scat(src_hbm)

    @pl.when(c == 1)
    def _():
        scat(dst_hbm)

    pltpu.sync_copy(hist_v, deg_sh.at[ident_v], add=True)
    plsc.subcore_barrier()

    @pl.when(s < HR // 8)
    def _():
        pltpu.sync_copy(deg_sh.at[pl.ds(rb, 8)], hist_v.at[pl.ds(0, 8)])
        pltpu.sync_copy(hist_v.at[pl.ds(0, 8)], deg_hbm.at[c, pl.ds(rb, 8)])


@jax.jit
def _deg_call(src_flat, dst_flat, ident):
    f = pl.kernel(
        _deg_body,
        out_type=jax.ShapeDtypeStruct((NC, HR, 128), jnp.float32),
        mesh=_mesh,
        compiler_params=_no_layout,
        scratch_types=[
            pltpu.VMEM_SHARED((HR, 128), jnp.float32),
            pltpu.VMEM((HR, 128), jnp.float32),
            pltpu.VMEM((EBLK,), jnp.int32),
            pltpu.VMEM((HR,), jnp.int32),
        ],
    )
    return f(src_flat, dst_flat, ident)


def _norm_body(x_ref, dego_ref, degi_ref, xs0_ref, xs1_ref, ns_ref, nd_ref):
    ns = lax.rsqrt(jnp.clip(dego_ref[...], 1.0, None))
    nd = lax.rsqrt(jnp.clip(degi_ref[...], 1.0, None))
    ns_ref[...] = ns
    nd_ref[...] = nd
    xs = x_ref[...] * ns
    xs0_ref[...] = xs[:, :FH]
    xs1_ref[...] = xs[:, FH:]


_norm_call = pl.pallas_call(
    _norm_body,
    grid=(NPAD // BK,),
    in_specs=[
        pl.BlockSpec((BK, F), lambda i: (i, 0)),
        pl.BlockSpec((BK, 1), lambda i: (i, 0)),
        pl.BlockSpec((BK, 1), lambda i: (i, 0)),
    ],
    out_specs=[
        pl.BlockSpec((BK, FH), lambda i: (i, 0)),
        pl.BlockSpec((BK, FH), lambda i: (i, 0)),
        pl.BlockSpec((BK, 1), lambda i: (i, 0)),
        pl.BlockSpec((BK, 1), lambda i: (i, 0)),
    ],
    out_shape=[
        jax.ShapeDtypeStruct((NPAD, FH), jnp.float32),
        jax.ShapeDtypeStruct((NPAD, FH), jnp.float32),
        jax.ShapeDtypeStruct((NPAD, 1), jnp.float32),
        jax.ShapeDtypeStruct((NPAD, 1), jnp.float32),
    ],
)


def _agg_body(xs0_hbm, xs1_hbm, src_hbm, dst_hbm, zeros_hbm,
              s0_hbm, s1_hbm, S_sh, sidx_v, didx_v, sidx2_v, didx2_v,
              isem_a, isem_b, *ring):
    c = lax.axis_index("c")
    s = lax.axis_index("s")
    zb = s * ZROWS
    bufs = list(ring[:SNBUF])
    gsem = list(ring[SNBUF:2 * SNBUF])
    ssem = list(ring[2 * SNBUF:3 * SNBUF])
    g0 = bufs[0]

    pltpu.sync_copy(zeros_hbm, g0)

    def zero_step(k, _):
        pltpu.sync_copy(g0, S_sh.at[pl.ds(zb + k * SCC, SCC)])
        return 0
    lax.fori_loop(0, ZROWS // SCC, zero_step, 0)
    plsc.subcore_barrier()

    rowb = s * STILE
    NBLK = STILE // SIB

    def s_phase(xs_hbm):
        def drain_scat(b, rowref):
            pltpu.make_async_copy(bufs[b], S_sh.at[rowref], ssem[b]).wait()

        def drain_idx(isem, rows, sref, dref):
            pltpu.make_async_copy(src_hbm.at[rows], sref, isem).wait()
            pltpu.make_async_copy(dst_hbm.at[rows], dref, isem).wait()

        def blk_body(bid, scur, dcur, snext, dnext, isem_cur, isem_next,
                     drain_cur_pred, prefetch_pred):
            @pl.when(drain_cur_pred)
            def _():
                drain_idx(isem_cur, pl.ds(rowb + bid * SIB, SIB), scur, dcur)

            gd = [None] * SNBUF
            for k in range(SIB):
                b = k % SNBUF
                if k >= SNBUF:
                    drain_scat(b, dcur.at[0])
                else:
                    @pl.when(bid > 0)
                    def _(b=b):
                        drain_scat(b, dcur.at[0])
                if k == SNBUF:
                    @pl.when(prefetch_pred)
                    def _():
                        rows = pl.ds(rowb + (bid + 1) * SIB, SIB)
                        pltpu.async_copy(src_hbm.at[rows], snext, isem_next)
                        pltpu.async_copy(dst_hbm.at[rows], dnext, isem_next)
                gd[b] = pltpu.async_copy(xs_hbm.at[scur.at[k]], bufs[b], gsem[b])
                if k >= 1:
                    pb = (k - 1) % SNBUF
                    gd[pb].wait()
                    pltpu.async_copy(bufs[pb], S_sh.at[dcur.at[k - 1]],
                                     ssem[pb], add=True)
            lb = (SIB - 1) % SNBUF
            gd[lb].wait()
            pltpu.async_copy(bufs[lb], S_sh.at[dcur.at[SIB - 1]], ssem[lb],
                             add=True)

        pltpu.sync_copy(src_hbm.at[pl.ds(rowb, SIB)], sidx_v)
        pltpu.sync_copy(dst_hbm.at[pl.ds(rowb, SIB)], didx_v)

        def blkpair(bp, _):
            b0 = 2 * bp
            blk_body(b0, sidx_v, didx_v, sidx2_v, didx2_v, isem_a, isem_b,
                     bp > 0, b0 + 1 < NBLK)
            blk_body(b0 + 1, sidx2_v, didx2_v, sidx_v, didx_v, isem_b, isem_a,
                     b0 + 1 < NBLK, b0 + 2 < NBLK)
            return 0
        lax.fori_loop(0, NBLK // 2, blkpair, 0)
        for b in range(SNBUF):
            drain_scat(b, didx_v.at[0])

    @pl.when(c == 0)
    def _():
        s_phase(xs0_hbm)

    @pl.when(c == 1)
    def _():
        s_phase(xs1_hbm)

    plsc.subcore_barrier()

    def wout_step(k, _):
        pltpu.sync_copy(S_sh.at[pl.ds(zb + k * SCC, SCC)], g0)

        @pl.when(c == 0)
        def _():
            pltpu.sync_copy(g0, s0_hbm.at[pl.ds(zb + k * SCC, SCC)])

        @pl.when(c == 1)
        def _():
            pltpu.sync_copy(g0, s1_hbm.at[pl.ds(zb + k * SCC, SCC)])

        return 0
    lax.fori_loop(0, ZROWS // SCC, wout_step, 0)


@jax.jit
def _agg_call(xs0, xs1, src2d, dst2d, zeros64):
    f = pl.kernel(
        _agg_body,
        out_type=[
            jax.ShapeDtypeStruct((NPAD, FH), jnp.float32),
            jax.ShapeDtypeStruct((NPAD, FH), jnp.float32),
        ],
        mesh=_mesh,
        scratch_types=(
            [
                pltpu.VMEM_SHARED((NPAD, FH), jnp.float32),
                pltpu.VMEM((SIB, SCC), jnp.int32),
                pltpu.VMEM((SIB, SCC), jnp.int32),
                pltpu.VMEM((SIB, SCC), jnp.int32),
                pltpu.VMEM((SIB, SCC), jnp.int32),
                pltpu.SemaphoreType.DMA,
                pltpu.SemaphoreType.DMA,
            ]
            + [pltpu.VMEM((SCC, FH), jnp.float32)] * SNBUF
            + [pltpu.SemaphoreType.DMA] * (2 * SNBUF)
        ),
    )
    return f(xs0, xs1, src2d, dst2d, zeros64)


def _r_body(nd_hbm, src_hbm, dst_hbm, ident_hbm, rp_hbm,
            r_sh, hist_v, nd_v, sidx_v, didx_v, ident_v):
    c = lax.axis_index("c")
    s = lax.axis_index("s")
    w = s * NC + c

    def zrow(i, _):
        for g in range(8):
            hist_v[i, pl.ds(g * 16, 16)] = jnp.zeros((16,), jnp.float32)
        return 0
    lax.fori_loop(0, HR, zrow, 0)

    rb = s * 8

    @pl.when(s < HR // 8)
    def _():
        pltpu.sync_copy(hist_v.at[pl.ds(0, 8)], r_sh.at[pl.ds(rb, 8)])

    pltpu.sync_copy(nd_hbm, nd_v)
    pltpu.sync_copy(ident_hbm, ident_v)
    plsc.subcore_barrier()

    ebase = w * ERT

    def blk(b, _):
        pltpu.sync_copy(src_hbm.at[pl.ds(ebase + b * EBLK, EBLK)], sidx_v)
        pltpu.sync_copy(dst_hbm.at[pl.ds(ebase + b * EBLK, EBLK)], didx_v)
        def grp(g, _):
            s16 = sidx_v[pl.ds(g * 16, 16)]
            d16 = didx_v[pl.ds(g * 16, 16)]
            vals = plsc.load_gather(nd_v, [d16])
            hi = jnp.right_shift(s16, 7)
            lo = jnp.bitwise_and(s16, 127)
            plsc.addupdate_scatter(hist_v, [hi, lo], vals)
            return 0
        lax.fori_loop(0, EBLK // 16, grp, 0)
        return 0
    lax.fori_loop(0, ERT // EBLK, blk, 0)

    pltpu.sync_copy(hist_v, r_sh.at[ident_v], add=True)
    plsc.subcore_barrier()

    @pl.when(s < HR // 8)
    def _():
        pltpu.sync_copy(r_sh.at[pl.ds(rb, 8)], hist_v.at[pl.ds(0, 8)])
        pltpu.sync_copy(hist_v.at[pl.ds(0, 8)], rp_hbm.at[c, pl.ds(rb, 8)])


@jax.jit
def _r_call(nd_flat, src_flat, dst_flat, ident):
    f = pl.kernel(
        _r_body,
        out_type=jax.ShapeDtypeStruct((NC, HR, 128), jnp.float32),
        mesh=_mesh,
        compiler_params=_no_layout,
        scratch_types=[
            pltpu.VMEM_SHARED((HR, 128), jnp.float32),
            pltpu.VMEM((HR, 128), jnp.float32),
            pltpu.VMEM((NPAD,), jnp.float32),
            pltpu.VMEM((EBLK,), jnp.int32),
            pltpu.VMEM((EBLK,), jnp.int32),
            pltpu.VMEM((HR,), jnp.int32),
        ],
    )
    return f(nd_flat, src_flat, dst_flat, ident)


def _final_body(s0_ref, s1_ref, nd_ref, ns_ref, rp0_ref, rp1_ref,
                w1_ref, b1_ref, w2_ref, b2_ref, out_ref, acc):
    i = pl.program_id(0)

    @pl.when(i == 0)
    def _():
        acc[...] = jnp.zeros_like(acc)

    sblk = jnp.concatenate([s0_ref[...], s1_ref[...]], axis=1)
    o = jnp.dot(sblk, w1_ref[...], preferred_element_type=jnp.float32)
    h = jnp.maximum(o * nd_ref[...] + b1_ref[...], 0.0)
    r = rp0_ref[...] + rp1_ref[...]
    cvec = ns_ref[...] * r
    rowid = i * BK + lax.broadcasted_iota(jnp.int32, (BK, 1), 0)
    cvec = jnp.where(rowid < N, cvec, 0.0)
    acc[...] += lax.dot_general(cvec, h, (((0,), (0,)), ((), ())),
                                preferred_element_type=jnp.float32)

    @pl.when(i == NPAD // BK - 1)
    def _():
        out_ref[...] = jnp.dot(acc[...] * (1.0 / N), w2_ref[...],
                               preferred_element_type=jnp.float32) + b2_ref[...]


_final_call = pl.pallas_call(
    _final_body,
    grid=(NPAD // BK,),
    in_specs=[
        pl.BlockSpec((BK, FH), lambda i: (i, 0)),
        pl.BlockSpec((BK, FH), lambda i: (i, 0)),
        pl.BlockSpec((BK, 1), lambda i: (i, 0)),
        pl.BlockSpec((BK, 1), lambda i: (i, 0)),
        pl.BlockSpec((BK, 1), lambda i: (i, 0)),
        pl.BlockSpec((BK, 1), lambda i: (i, 0)),
        pl.BlockSpec((F, F), lambda i: (0, 0)),
        pl.BlockSpec((1, F), lambda i: (0, 0)),
        pl.BlockSpec((F, NCLS), lambda i: (0, 0)),
        pl.BlockSpec((1, NCLS), lambda i: (0, 0)),
    ],
    out_specs=pl.BlockSpec((1, NCLS), lambda i: (0, 0)),
    out_shape=jax.ShapeDtypeStruct((1, NCLS), jnp.float32),
    scratch_shapes=[pltpu.VMEM((1, F), jnp.float32)],
)


def kernel(x, edge_index, W1, b1, W2, b2):
    src = edge_index[0].astype(jnp.int32)
    dst = edge_index[1].astype(jnp.int32)
    pad = jnp.full((EP - E,), DUMMY, jnp.int32)
    src_flat = jnp.concatenate([src, pad])
    dst_flat = jnp.concatenate([dst, pad])
    src2d = src_flat.reshape(SNCHUNK, SCC)
    dst2d = dst_flat.reshape(SNCHUNK, SCC)
    x_pad = jnp.pad(x, ((0, NPAD - N), (0, 0)))

    ident = jnp.arange(HR, dtype=jnp.int32)
    zeros64 = jnp.zeros((SCC, FH), jnp.float32)

    degs = _deg_call(src_flat, dst_flat, ident)
    dego_col = degs[0].reshape(NPAD, 1)
    degi_col = degs[1].reshape(NPAD, 1)
    xs0, xs1, ns_col, nd_col = _norm_call(x_pad, dego_col, degi_col)
    s0, s1 = _agg_call(xs0, xs1, src2d, dst2d, zeros64)
    rp = _r_call(nd_col.reshape(NPAD), src_flat, dst_flat, ident)
    out = _final_call(s0, s1, nd_col, ns_col,
                      rp[0].reshape(NPAD, 1), rp[1].reshape(NPAD, 1),
                      W1, b1.reshape(1, F), W2, b2.reshape(1, NCLS))
    return out


# speedup vs baseline: 1.0802x; 1.0270x over previous
import jax
import jax.numpy as jnp
from jax import lax
from jax.experimental import pallas as pl
from jax.experimental.pallas import tpu as pltpu
from jax.experimental.pallas import tpu_sc as plsc

N = 10000
E = 160000
F = 256
FH = 128
NCLS = 64
NC, NS = 2, 16
NW = NC * NS
CH = 128
IBLK = 8
NPAD = 10240
DUMMY = N
EP = 163840
NCHUNK = EP // CH
SCHUNK = NCHUNK // NS
SCC = 64
SNCHUNK = EP // SCC
STILE = SNCHUNK // NS
SIB = 16
SNBUF = 4
SLAG = 2
ZROWS = NPAD // NS
HR = NPAD // 128
EBLK = 1024
EDT = EP // NS
ERT = EP // NW
BK = 2048

_mesh = plsc.VectorSubcoreMesh(core_axis_name="c", subcore_axis_name="s")
_no_layout = pltpu.CompilerParams(needs_layout_passes=False)


def _deg_body(src_hbm, dst_hbm, ident_hbm, deg_hbm,
              deg_sh, hist_v, idx_v, ident_v):
    c = lax.axis_index("c")
    s = lax.axis_index("s")

    def zrow(i, _):
        for g in range(8):
            hist_v[i, pl.ds(g * 16, 16)] = jnp.zeros((16,), jnp.float32)
        return 0
    lax.fori_loop(0, HR, zrow, 0)

    rb = s * 8

    @pl.when(s < HR // 8)
    def _():
        pltpu.sync_copy(hist_v.at[pl.ds(0, 8)], deg_sh.at[pl.ds(rb, 8)])

    pltpu.sync_copy(ident_hbm, ident_v)
    plsc.subcore_barrier()

    ebase = s * EDT
    ones16 = jnp.full((16,), 1.0, jnp.float32)

    def scat(idx_hbm):
        def blk(b, _):
            pltpu.sync_copy(idx_hbm.at[pl.ds(ebase + b * EBLK, EBLK)], idx_v)
            def grp(g, _):
                idx16 = idx_v[pl.ds(g * 16, 16)]
                hi = jnp.right_shift(idx16, 7)
                lo = jnp.bitwise_and(idx16, 127)
                plsc.addupdate_scatter(hist_v, [hi, lo], ones16)
                return 0
            lax.fori_loop(0, EBLK // 16, grp, 0)
            return 0
        lax.fori_loop(0, EDT // EBLK, blk, 0)

    @pl.when(c == 0)
    def _():
        scat(src_hbm)

    @pl.when(c == 1)
    def _():
        scat(dst_hbm)

    pltpu.sync_copy(hist_v, deg_sh.at[ident_v], add=True)
    plsc.subcore_barrier()

    @pl.when(s < HR // 8)
    def _():
        pltpu.sync_copy(deg_sh.at[pl.ds(rb, 8)], hist_v.at[pl.ds(0, 8)])
        pltpu.sync_copy(hist_v.at[pl.ds(0, 8)], deg_hbm.at[c, pl.ds(rb, 8)])


@jax.jit
def _deg_call(src_flat, dst_flat, ident):
    f = pl.kernel(
        _deg_body,
        out_type=jax.ShapeDtypeStruct((NC, HR, 128), jnp.float32),
        mesh=_mesh,
        compiler_params=_no_layout,
        scratch_types=[
            pltpu.VMEM_SHARED((HR, 128), jnp.float32),
            pltpu.VMEM((HR, 128), jnp.float32),
            pltpu.VMEM((EBLK,), jnp.int32),
            pltpu.VMEM((HR,), jnp.int32),
        ],
    )
    return f(src_flat, dst_flat, ident)


def _norm_body(x_ref, dego_ref, degi_ref, xs0_ref, xs1_ref, ns_ref, nd_ref):
    ns = lax.rsqrt(jnp.clip(dego_ref[...], 1.0, None))
    nd = lax.rsqrt(jnp.clip(degi_ref[...], 1.0, None))
    ns_ref[...] = ns
    nd_ref[...] = nd
    xs = x_ref[...] * ns
    xs0_ref[...] = xs[:, :FH]
    xs1_ref[...] = xs[:, FH:]


_norm_call = pl.pallas_call(
    _norm_body,
    grid=(NPAD // BK,),
    in_specs=[
        pl.BlockSpec((BK, F), lambda i: (i, 0)),
        pl.BlockSpec((BK, 1), lambda i: (i, 0)),
        pl.BlockSpec((BK, 1), lambda i: (i, 0)),
    ],
    out_specs=[
        pl.BlockSpec((BK, FH), lambda i: (i, 0)),
        pl.BlockSpec((BK, FH), lambda i: (i, 0)),
        pl.BlockSpec((BK, 1), lambda i: (i, 0)),
        pl.BlockSpec((BK, 1), lambda i: (i, 0)),
    ],
    out_shape=[
        jax.ShapeDtypeStruct((NPAD, FH), jnp.float32),
        jax.ShapeDtypeStruct((NPAD, FH), jnp.float32),
        jax.ShapeDtypeStruct((NPAD, 1), jnp.float32),
        jax.ShapeDtypeStruct((NPAD, 1), jnp.float32),
    ],
)


def _agg_body(xs0_hbm, xs1_hbm, src_hbm, dst_hbm, zeros_hbm,
              s0_hbm, s1_hbm, S_sh, sidx_v, didx_v, sidx2_v, didx2_v,
              isem_a, isem_b, *ring):
    c = lax.axis_index("c")
    s = lax.axis_index("s")
    zb = s * ZROWS
    bufs = list(ring[:SNBUF])
    gsem = list(ring[SNBUF:2 * SNBUF])
    ssem = list(ring[2 * SNBUF:3 * SNBUF])
    g0 = bufs[0]

    pltpu.sync_copy(zeros_hbm, g0)

    def zero_step(k, _):
        pltpu.sync_copy(g0, S_sh.at[pl.ds(zb + k * SCC, SCC)])
        return 0
    lax.fori_loop(0, ZROWS // SCC, zero_step, 0)
    plsc.subcore_barrier()

    rowb = s * STILE
    NBLK = STILE // SIB

    def s_phase(xs_hbm):
        def drain_scat(b, rowref):
            pltpu.make_async_copy(bufs[b], S_sh.at[rowref], ssem[b]).wait()

        def drain_idx(isem, rows, sref, dref):
            pltpu.make_async_copy(src_hbm.at[rows], sref, isem).wait()
            pltpu.make_async_copy(dst_hbm.at[rows], dref, isem).wait()

        def blk_body(bid, scur, dcur, snext, dnext, isem_cur, isem_next,
                     drain_cur_pred, prefetch_pred):
            @pl.when(drain_cur_pred)
            def _():
                drain_idx(isem_cur, pl.ds(rowb + bid * SIB, SIB), scur, dcur)

            gd = [None] * SNBUF
            for k in range(SIB):
                b = k % SNBUF
                if k >= SNBUF:
                    drain_scat(b, dcur.at[0])
                else:
                    @pl.when(bid > 0)
                    def _(b=b):
                        drain_scat(b, dcur.at[0])
                if k == SNBUF:
                    @pl.when(prefetch_pred)
                    def _():
                        rows = pl.ds(rowb + (bid + 1) * SIB, SIB)
                        pltpu.async_copy(src_hbm.at[rows], snext, isem_next)
                        pltpu.async_copy(dst_hbm.at[rows], dnext, isem_next)
                gd[b] = pltpu.async_copy(xs_hbm.at[scur.at[k]], bufs[b], gsem[b])
                if k >= SLAG:
                    pb = (k - SLAG) % SNBUF
                    gd[pb].wait()
                    pltpu.async_copy(bufs[pb], S_sh.at[dcur.at[k - SLAG]],
                                     ssem[pb], add=True)
            for t in range(SLAG):
                kk = SIB - SLAG + t
                pb = kk % SNBUF
                gd[pb].wait()
                pltpu.async_copy(bufs[pb], S_sh.at[dcur.at[kk]], ssem[pb],
                                 add=True)

        pltpu.sync_copy(src_hbm.at[pl.ds(rowb, SIB)], sidx_v)
        pltpu.sync_copy(dst_hbm.at[pl.ds(rowb, SIB)], didx_v)

        def blkpair(bp, _):
            b0 = 2 * bp
            blk_body(b0, sidx_v, didx_v, sidx2_v, didx2_v, isem_a, isem_b,
                     bp > 0, b0 + 1 < NBLK)
            blk_body(b0 + 1, sidx2_v, didx2_v, sidx_v, didx_v, isem_b, isem_a,
                     b0 + 1 < NBLK, b0 + 2 < NBLK)
            return 0
        lax.fori_loop(0, NBLK // 2, blkpair, 0)
        for b in range(SNBUF):
            drain_scat(b, didx_v.at[0])

    @pl.when(c == 0)
    def _():
        s_phase(xs0_hbm)

    @pl.when(c == 1)
    def _():
        s_phase(xs1_hbm)

    plsc.subcore_barrier()

    def wout_step(k, _):
        pltpu.sync_copy(S_sh.at[pl.ds(zb + k * SCC, SCC)], g0)

        @pl.when(c == 0)
        def _():
            pltpu.sync_copy(g0, s0_hbm.at[pl.ds(zb + k * SCC, SCC)])

        @pl.when(c == 1)
        def _():
            pltpu.sync_copy(g0, s1_hbm.at[pl.ds(zb + k * SCC, SCC)])

        return 0
    lax.fori_loop(0, ZROWS // SCC, wout_step, 0)


@jax.jit
def _agg_call(xs0, xs1, src2d, dst2d, zeros64):
    f = pl.kernel(
        _agg_body,
        out_type=[
            jax.ShapeDtypeStruct((NPAD, FH), jnp.float32),
            jax.ShapeDtypeStruct((NPAD, FH), jnp.float32),
        ],
        mesh=_mesh,
        scratch_types=(
            [
                pltpu.VMEM_SHARED((NPAD, FH), jnp.float32),
                pltpu.VMEM((SIB, SCC), jnp.int32),
                pltpu.VMEM((SIB, SCC), jnp.int32),
                pltpu.VMEM((SIB, SCC), jnp.int32),
                pltpu.VMEM((SIB, SCC), jnp.int32),
                pltpu.SemaphoreType.DMA,
                pltpu.SemaphoreType.DMA,
            ]
            + [pltpu.VMEM((SCC, FH), jnp.float32)] * SNBUF
            + [pltpu.SemaphoreType.DMA] * (2 * SNBUF)
        ),
    )
    return f(xs0, xs1, src2d, dst2d, zeros64)


def _r_body(nd_hbm, src_hbm, dst_hbm, ident_hbm, rp_hbm,
            r_sh, hist_v, nd_v, sidx_v, didx_v, ident_v):
    c = lax.axis_index("c")
    s = lax.axis_index("s")
    w = s * NC + c

    def zrow(i, _):
        for g in range(8):
            hist_v[i, pl.ds(g * 16, 16)] = jnp.zeros((16,), jnp.float32)
        return 0
    lax.fori_loop(0, HR, zrow, 0)

    rb = s * 8

    @pl.when(s < HR // 8)
    def _():
        pltpu.sync_copy(hist_v.at[pl.ds(0, 8)], r_sh.at[pl.ds(rb, 8)])

    pltpu.sync_copy(nd_hbm, nd_v)
    pltpu.sync_copy(ident_hbm, ident_v)
    plsc.subcore_barrier()

    ebase = w * ERT

    def blk(b, _):
        pltpu.sync_copy(src_hbm.at[pl.ds(ebase + b * EBLK, EBLK)], sidx_v)
        pltpu.sync_copy(dst_hbm.at[pl.ds(ebase + b * EBLK, EBLK)], didx_v)
        def grp(g, _):
            s16 = sidx_v[pl.ds(g * 16, 16)]
            d16 = didx_v[pl.ds(g * 16, 16)]
            vals = plsc.load_gather(nd_v, [d16])
            hi = jnp.right_shift(s16, 7)
            lo = jnp.bitwise_and(s16, 127)
            plsc.addupdate_scatter(hist_v, [hi, lo], vals)
            return 0
        lax.fori_loop(0, EBLK // 16, grp, 0)
        return 0
    lax.fori_loop(0, ERT // EBLK, blk, 0)

    pltpu.sync_copy(hist_v, r_sh.at[ident_v], add=True)
    plsc.subcore_barrier()

    @pl.when(s < HR // 8)
    def _():
        pltpu.sync_copy(r_sh.at[pl.ds(rb, 8)], hist_v.at[pl.ds(0, 8)])
        pltpu.sync_copy(hist_v.at[pl.ds(0, 8)], rp_hbm.at[c, pl.ds(rb, 8)])


@jax.jit
def _r_call(nd_flat, src_flat, dst_flat, ident):
    f = pl.kernel(
        _r_body,
        out_type=jax.ShapeDtypeStruct((NC, HR, 128), jnp.float32),
        mesh=_mesh,
        compiler_params=_no_layout,
        scratch_types=[
            pltpu.VMEM_SHARED((HR, 128), jnp.float32),
            pltpu.VMEM((HR, 128), jnp.float32),
            pltpu.VMEM((NPAD,), jnp.float32),
            pltpu.VMEM((EBLK,), jnp.int32),
            pltpu.VMEM((EBLK,), jnp.int32),
            pltpu.VMEM((HR,), jnp.int32),
        ],
    )
    return f(nd_flat, src_flat, dst_flat, ident)


def _final_body(s0_ref, s1_ref, nd_ref, ns_ref, rp0_ref, rp1_ref,
                w1_ref, b1_ref, w2_ref, b2_ref, out_ref, acc):
    i = pl.program_id(0)

    @pl.when(i == 0)
    def _():
        acc[...] = jnp.zeros_like(acc)

    sblk = jnp.concatenate([s0_ref[...], s1_ref[...]], axis=1)
    o = jnp.dot(sblk, w1_ref[...], preferred_element_type=jnp.float32)
    h = jnp.maximum(o * nd_ref[...] + b1_ref[...], 0.0)
    r = rp0_ref[...] + rp1_ref[...]
    cvec = ns_ref[...] * r
    rowid = i * BK + lax.broadcasted_iota(jnp.int32, (BK, 1), 0)
    cvec = jnp.where(rowid < N, cvec, 0.0)
    acc[...] += lax.dot_general(cvec, h, (((0,), (0,)), ((), ())),
                                preferred_element_type=jnp.float32)

    @pl.when(i == NPAD // BK - 1)
    def _():
        out_ref[...] = jnp.dot(acc[...] * (1.0 / N), w2_ref[...],
                               preferred_element_type=jnp.float32) + b2_ref[...]


_final_call = pl.pallas_call(
    _final_body,
    grid=(NPAD // BK,),
    in_specs=[
        pl.BlockSpec((BK, FH), lambda i: (i, 0)),
        pl.BlockSpec((BK, FH), lambda i: (i, 0)),
        pl.BlockSpec((BK, 1), lambda i: (i, 0)),
        pl.BlockSpec((BK, 1), lambda i: (i, 0)),
        pl.BlockSpec((BK, 1), lambda i: (i, 0)),
        pl.BlockSpec((BK, 1), lambda i: (i, 0)),
        pl.BlockSpec((F, F), lambda i: (0, 0)),
        pl.BlockSpec((1, F), lambda i: (0, 0)),
        pl.BlockSpec((F, NCLS), lambda i: (0, 0)),
        pl.BlockSpec((1, NCLS), lambda i: (0, 0)),
    ],
    out_specs=pl.BlockSpec((1, NCLS), lambda i: (0, 0)),
    out_shape=jax.ShapeDtypeStruct((1, NCLS), jnp.float32),
    scratch_shapes=[pltpu.VMEM((1, F), jnp.float32)],
)


def kernel(x, edge_index, W1, b1, W2, b2):
    src = edge_index[0].astype(jnp.int32)
    dst = edge_index[1].astype(jnp.int32)
    pad = jnp.full((EP - E,), DUMMY, jnp.int32)
    src_flat = jnp.concatenate([src, pad])
    dst_flat = jnp.concatenate([dst, pad])
    src2d = src_flat.reshape(SNCHUNK, SCC)
    dst2d = dst_flat.reshape(SNCHUNK, SCC)
    x_pad = jnp.pad(x, ((0, NPAD - N), (0, 0)))

    ident = jnp.arange(HR, dtype=jnp.int32)
    zeros64 = jnp.zeros((SCC, FH), jnp.float32)

    degs = _deg_call(src_flat, dst_flat, ident)
    dego_col = degs[0].reshape(NPAD, 1)
    degi_col = degs[1].reshape(NPAD, 1)
    xs0, xs1, ns_col, nd_col = _norm_call(x_pad, dego_col, degi_col)
    s0, s1 = _agg_call(xs0, xs1, src2d, dst2d, zeros64)
    rp = _r_call(nd_col.reshape(NPAD), src_flat, dst_flat, ident)
    out = _final_call(s0, s1, nd_col, ns_col,
                      rp[0].reshape(NPAD, 1), rp[1].reshape(NPAD, 1),
                      W1, b1.reshape(1, F), W2, b2.reshape(1, NCLS))
    return out


# speedup vs baseline: 1.0992x; 1.0176x over previous
import jax
import jax.numpy as jnp
from jax import lax
from jax.experimental import pallas as pl
from jax.experimental.pallas import tpu as pltpu
from jax.experimental.pallas import tpu_sc as plsc

N = 10000
E = 160000
F = 256
FH = 128
NCLS = 64
NC, NS = 2, 16
NW = NC * NS
CH = 128
IBLK = 8
NPAD = 10240
DUMMY = N
EP = 163840
NCHUNK = EP // CH
SCHUNK = NCHUNK // NS
SCC = 64
SNCHUNK = EP // SCC
STILE = SNCHUNK // NS
SIB = 16
SNBUF = 4
SLAG = 3
ZROWS = NPAD // NS
HR = NPAD // 128
EBLK = 1024
EDT = EP // NS
ERT = EP // NW
BK = 2048

_mesh = plsc.VectorSubcoreMesh(core_axis_name="c", subcore_axis_name="s")
_no_layout = pltpu.CompilerParams(needs_layout_passes=False)


def _deg_body(src_hbm, dst_hbm, ident_hbm, deg_hbm,
              deg_sh, hist_v, idx_v, ident_v):
    c = lax.axis_index("c")
    s = lax.axis_index("s")

    def zrow(i, _):
        for g in range(8):
            hist_v[i, pl.ds(g * 16, 16)] = jnp.zeros((16,), jnp.float32)
        return 0
    lax.fori_loop(0, HR, zrow, 0)

    rb = s * 8

    @pl.when(s < HR // 8)
    def _():
        pltpu.sync_copy(hist_v.at[pl.ds(0, 8)], deg_sh.at[pl.ds(rb, 8)])

    pltpu.sync_copy(ident_hbm, ident_v)
    plsc.subcore_barrier()

    ebase = s * EDT
    ones16 = jnp.full((16,), 1.0, jnp.float32)

    def scat(idx_hbm):
        def blk(b, _):
            pltpu.sync_copy(idx_hbm.at[pl.ds(ebase + b * EBLK, EBLK)], idx_v)
            def grp(g, _):
                idx16 = idx_v[pl.ds(g * 16, 16)]
                hi = jnp.right_shift(idx16, 7)
                lo = jnp.bitwise_and(idx16, 127)
                plsc.addupdate_scatter(hist_v, [hi, lo], ones16)
                return 0
            lax.fori_loop(0, EBLK // 16, grp, 0)
            return 0
        lax.fori_loop(0, EDT // EBLK, blk, 0)

    @pl.when(c == 0)
    def _():
        scat(src_hbm)

    @pl.when(c == 1)
    def _():
        scat(dst_hbm)

    pltpu.sync_copy(hist_v, deg_sh.at[ident_v], add=True)
    plsc.subcore_barrier()

    @pl.when(s < HR // 8)
    def _():
        pltpu.sync_copy(deg_sh.at[pl.ds(rb, 8)], hist_v.at[pl.ds(0, 8)])
        pltpu.sync_copy(hist_v.at[pl.ds(0, 8)], deg_hbm.at[c, pl.ds(rb, 8)])


@jax.jit
def _deg_call(src_flat, dst_flat, ident):
    f = pl.kernel(
        _deg_body,
        out_type=jax.ShapeDtypeStruct((NC, HR, 128), jnp.float32),
        mesh=_mesh,
        compiler_params=_no_layout,
        scratch_types=[
            pltpu.VMEM_SHARED((HR, 128), jnp.float32),
            pltpu.VMEM((HR, 128), jnp.float32),
            pltpu.VMEM((EBLK,), jnp.int32),
            pltpu.VMEM((HR,), jnp.int32),
        ],
    )
    return f(src_flat, dst_flat, ident)


def _norm_body(x_ref, dego_ref, degi_ref, xs0_ref, xs1_ref, ns_ref, nd_ref):
    ns = lax.rsqrt(jnp.clip(dego_ref[...], 1.0, None))
    nd = lax.rsqrt(jnp.clip(degi_ref[...], 1.0, None))
    ns_ref[...] = ns
    nd_ref[...] = nd
    xs = x_ref[...] * ns
    xs0_ref[...] = xs[:, :FH]
    xs1_ref[...] = xs[:, FH:]


_norm_call = pl.pallas_call(
    _norm_body,
    grid=(NPAD // BK,),
    in_specs=[
        pl.BlockSpec((BK, F), lambda i: (i, 0)),
        pl.BlockSpec((BK, 1), lambda i: (i, 0)),
        pl.BlockSpec((BK, 1), lambda i: (i, 0)),
    ],
    out_specs=[
        pl.BlockSpec((BK, FH), lambda i: (i, 0)),
        pl.BlockSpec((BK, FH), lambda i: (i, 0)),
        pl.BlockSpec((BK, 1), lambda i: (i, 0)),
        pl.BlockSpec((BK, 1), lambda i: (i, 0)),
    ],
    out_shape=[
        jax.ShapeDtypeStruct((NPAD, FH), jnp.float32),
        jax.ShapeDtypeStruct((NPAD, FH), jnp.float32),
        jax.ShapeDtypeStruct((NPAD, 1), jnp.float32),
        jax.ShapeDtypeStruct((NPAD, 1), jnp.float32),
    ],
)


def _agg_body(xs0_hbm, xs1_hbm, src_hbm, dst_hbm, zeros_hbm,
              s0_hbm, s1_hbm, S_sh, sidx_v, didx_v, sidx2_v, didx2_v,
              isem_a, isem_b, *ring):
    c = lax.axis_index("c")
    s = lax.axis_index("s")
    zb = s * ZROWS
    bufs = list(ring[:SNBUF])
    gsem = list(ring[SNBUF:2 * SNBUF])
    ssem = list(ring[2 * SNBUF:3 * SNBUF])
    g0 = bufs[0]

    pltpu.sync_copy(zeros_hbm, g0)

    def zero_step(k, _):
        pltpu.sync_copy(g0, S_sh.at[pl.ds(zb + k * SCC, SCC)])
        return 0
    lax.fori_loop(0, ZROWS // SCC, zero_step, 0)
    plsc.subcore_barrier()

    rowb = s * STILE
    NBLK = STILE // SIB

    def s_phase(xs_hbm):
        def drain_scat(b, rowref):
            pltpu.make_async_copy(bufs[b], S_sh.at[rowref], ssem[b]).wait()

        def drain_idx(isem, rows, sref, dref):
            pltpu.make_async_copy(src_hbm.at[rows], sref, isem).wait()
            pltpu.make_async_copy(dst_hbm.at[rows], dref, isem).wait()

        def blk_body(bid, scur, dcur, snext, dnext, isem_cur, isem_next,
                     drain_cur_pred, prefetch_pred):
            @pl.when(drain_cur_pred)
            def _():
                drain_idx(isem_cur, pl.ds(rowb + bid * SIB, SIB), scur, dcur)

            gd = [None] * SNBUF
            for k in range(SIB):
                b = k % SNBUF
                if k >= SNBUF:
                    drain_scat(b, dcur.at[0])
                else:
                    @pl.when(bid > 0)
                    def _(b=b):
                        drain_scat(b, dcur.at[0])
                if k == SNBUF:
                    @pl.when(prefetch_pred)
                    def _():
                        rows = pl.ds(rowb + (bid + 1) * SIB, SIB)
                        pltpu.async_copy(src_hbm.at[rows], snext, isem_next)
                        pltpu.async_copy(dst_hbm.at[rows], dnext, isem_next)
                gd[b] = pltpu.async_copy(xs_hbm.at[scur.at[k]], bufs[b], gsem[b])
                if k >= SLAG:
                    pb = (k - SLAG) % SNBUF
                    gd[pb].wait()
                    pltpu.async_copy(bufs[pb], S_sh.at[dcur.at[k - SLAG]],
                                     ssem[pb], add=True)
            for t in range(SLAG):
                kk = SIB - SLAG + t
                pb = kk % SNBUF
                gd[pb].wait()
                pltpu.async_copy(bufs[pb], S_sh.at[dcur.at[kk]], ssem[pb],
                                 add=True)

        pltpu.sync_copy(src_hbm.at[pl.ds(rowb, SIB)], sidx_v)
        pltpu.sync_copy(dst_hbm.at[pl.ds(rowb, SIB)], didx_v)

        def blkpair(bp, _):
            b0 = 2 * bp
            blk_body(b0, sidx_v, didx_v, sidx2_v, didx2_v, isem_a, isem_b,
                     bp > 0, b0 + 1 < NBLK)
            blk_body(b0 + 1, sidx2_v, didx2_v, sidx_v, didx_v, isem_b, isem_a,
                     b0 + 1 < NBLK, b0 + 2 < NBLK)
            return 0
        lax.fori_loop(0, NBLK // 2, blkpair, 0)
        for b in range(SNBUF):
            drain_scat(b, didx_v.at[0])

    @pl.when(c == 0)
    def _():
        s_phase(xs0_hbm)

    @pl.when(c == 1)
    def _():
        s_phase(xs1_hbm)

    plsc.subcore_barrier()

    def wout_step(k, _):
        pltpu.sync_copy(S_sh.at[pl.ds(zb + k * SCC, SCC)], g0)

        @pl.when(c == 0)
        def _():
            pltpu.sync_copy(g0, s0_hbm.at[pl.ds(zb + k * SCC, SCC)])

        @pl.when(c == 1)
        def _():
            pltpu.sync_copy(g0, s1_hbm.at[pl.ds(zb + k * SCC, SCC)])

        return 0
    lax.fori_loop(0, ZROWS // SCC, wout_step, 0)


@jax.jit
def _agg_call(xs0, xs1, src2d, dst2d, zeros64):
    f = pl.kernel(
        _agg_body,
        out_type=[
            jax.ShapeDtypeStruct((NPAD, FH), jnp.float32),
            jax.ShapeDtypeStruct((NPAD, FH), jnp.float32),
        ],
        mesh=_mesh,
        scratch_types=(
            [
                pltpu.VMEM_SHARED((NPAD, FH), jnp.float32),
                pltpu.VMEM((SIB, SCC), jnp.int32),
                pltpu.VMEM((SIB, SCC), jnp.int32),
                pltpu.VMEM((SIB, SCC), jnp.int32),
                pltpu.VMEM((SIB, SCC), jnp.int32),
                pltpu.SemaphoreType.DMA,
                pltpu.SemaphoreType.DMA,
            ]
            + [pltpu.VMEM((SCC, FH), jnp.float32)] * SNBUF
            + [pltpu.SemaphoreType.DMA] * (2 * SNBUF)
        ),
    )
    return f(xs0, xs1, src2d, dst2d, zeros64)


def _r_body(nd_hbm, src_hbm, dst_hbm, ident_hbm, rp_hbm,
            r_sh, hist_v, nd_v, sidx_v, didx_v, ident_v):
    c = lax.axis_index("c")
    s = lax.axis_index("s")
    w = s * NC + c

    def zrow(i, _):
        for g in range(8):
            hist_v[i, pl.ds(g * 16, 16)] = jnp.zeros((16,), jnp.float32)
        return 0
    lax.fori_loop(0, HR, zrow, 0)

    rb = s * 8

    @pl.when(s < HR // 8)
    def _():
        pltpu.sync_copy(hist_v.at[pl.ds(0, 8)], r_sh.at[pl.ds(rb, 8)])

    pltpu.sync_copy(nd_hbm, nd_v)
    pltpu.sync_copy(ident_hbm, ident_v)
    plsc.subcore_barrier()

    ebase = w * ERT

    def blk(b, _):
        pltpu.sync_copy(src_hbm.at[pl.ds(ebase + b * EBLK, EBLK)], sidx_v)
        pltpu.sync_copy(dst_hbm.at[pl.ds(ebase + b * EBLK, EBLK)], didx_v)
        def grp(g, _):
            s16 = sidx_v[pl.ds(g * 16, 16)]
            d16 = didx_v[pl.ds(g * 16, 16)]
            vals = plsc.load_gather(nd_v, [d16])
            hi = jnp.right_shift(s16, 7)
            lo = jnp.bitwise_and(s16, 127)
            plsc.addupdate_scatter(hist_v, [hi, lo], vals)
            return 0
        lax.fori_loop(0, EBLK // 16, grp, 0)
        return 0
    lax.fori_loop(0, ERT // EBLK, blk, 0)

    pltpu.sync_copy(hist_v, r_sh.at[ident_v], add=True)
    plsc.subcore_barrier()

    @pl.when(s < HR // 8)
    def _():
        pltpu.sync_copy(r_sh.at[pl.ds(rb, 8)], hist_v.at[pl.ds(0, 8)])
        pltpu.sync_copy(hist_v.at[pl.ds(0, 8)], rp_hbm.at[c, pl.ds(rb, 8)])


@jax.jit
def _r_call(nd_flat, src_flat, dst_flat, ident):
    f = pl.kernel(
        _r_body,
        out_type=jax.ShapeDtypeStruct((NC, HR, 128), jnp.float32),
        mesh=_mesh,
        compiler_params=_no_layout,
        scratch_types=[
            pltpu.VMEM_SHARED((HR, 128), jnp.float32),
            pltpu.VMEM((HR, 128), jnp.float32),
            pltpu.VMEM((NPAD,), jnp.float32),
            pltpu.VMEM((EBLK,), jnp.int32),
            pltpu.VMEM((EBLK,), jnp.int32),
            pltpu.VMEM((HR,), jnp.int32),
        ],
    )
    return f(nd_flat, src_flat, dst_flat, ident)


def _final_body(s0_ref, s1_ref, nd_ref, ns_ref, rp0_ref, rp1_ref,
                w1_ref, b1_ref, w2_ref, b2_ref, out_ref, acc):
    i = pl.program_id(0)

    @pl.when(i == 0)
    def _():
        acc[...] = jnp.zeros_like(acc)

    sblk = jnp.concatenate([s0_ref[...], s1_ref[...]], axis=1)
    o = jnp.dot(sblk, w1_ref[...], preferred_element_type=jnp.float32)
    h = jnp.maximum(o * nd_ref[...] + b1_ref[...], 0.0)
    r = rp0_ref[...] + rp1_ref[...]
    cvec = ns_ref[...] * r
    rowid = i * BK + lax.broadcasted_iota(jnp.int32, (BK, 1), 0)
    cvec = jnp.where(rowid < N, cvec, 0.0)
    acc[...] += lax.dot_general(cvec, h, (((0,), (0,)), ((), ())),
                                preferred_element_type=jnp.float32)

    @pl.when(i == NPAD // BK - 1)
    def _():
        out_ref[...] = jnp.dot(acc[...] * (1.0 / N), w2_ref[...],
                               preferred_element_type=jnp.float32) + b2_ref[...]


_final_call = pl.pallas_call(
    _final_body,
    grid=(NPAD // BK,),
    in_specs=[
        pl.BlockSpec((BK, FH), lambda i: (i, 0)),
        pl.BlockSpec((BK, FH), lambda i: (i, 0)),
        pl.BlockSpec((BK, 1), lambda i: (i, 0)),
        pl.BlockSpec((BK, 1), lambda i: (i, 0)),
        pl.BlockSpec((BK, 1), lambda i: (i, 0)),
        pl.BlockSpec((BK, 1), lambda i: (i, 0)),
        pl.BlockSpec((F, F), lambda i: (0, 0)),
        pl.BlockSpec((1, F), lambda i: (0, 0)),
        pl.BlockSpec((F, NCLS), lambda i: (0, 0)),
        pl.BlockSpec((1, NCLS), lambda i: (0, 0)),
    ],
    out_specs=pl.BlockSpec((1, NCLS), lambda i: (0, 0)),
    out_shape=jax.ShapeDtypeStruct((1, NCLS), jnp.float32),
    scratch_shapes=[pltpu.VMEM((1, F), jnp.float32)],
)


def kernel(x, edge_index, W1, b1, W2, b2):
    src = edge_index[0].astype(jnp.int32)
    dst = edge_index[1].astype(jnp.int32)
    pad = jnp.full((EP - E,), DUMMY, jnp.int32)
    src_flat = jnp.concatenate([src, pad])
    dst_flat = jnp.concatenate([dst, pad])
    src2d = src_flat.reshape(SNCHUNK, SCC)
    dst2d = dst_flat.reshape(SNCHUNK, SCC)
    x_pad = jnp.pad(x, ((0, NPAD - N), (0, 0)))

    ident = jnp.arange(HR, dtype=jnp.int32)
    zeros64 = jnp.zeros((SCC, FH), jnp.float32)

    degs = _deg_call(src_flat, dst_flat, ident)
    dego_col = degs[0].reshape(NPAD, 1)
    degi_col = degs[1].reshape(NPAD, 1)
    xs0, xs1, ns_col, nd_col = _norm_call(x_pad, dego_col, degi_col)
    s0, s1 = _agg_call(xs0, xs1, src2d, dst2d, zeros64)
    rp = _r_call(nd_col.reshape(NPAD), src_flat, dst_flat, ident)
    out = _final_call(s0, s1, nd_col, ns_col,
                      rp[0].reshape(NPAD, 1), rp[1].reshape(NPAD, 1),
                      W1, b1.reshape(1, F), W2, b2.reshape(1, NCLS))
    return out
